# Initial kernel scaffold; baseline (speedup 1.0000x reference)
import jax
import jax.numpy as jnp
from jax.experimental import pallas as pl


def kernel(x, table, W1, b1, W2, b2, Wo, bo):
    raise NotImplementedError("write your pallas kernel here")


# speedup vs baseline: 7.3339x; 7.3339x over previous
import functools

import jax
import jax.numpy as jnp
import numpy as np
from jax import lax
from jax.experimental import pallas as pl
from jax.experimental.pallas import tpu as pltpu
from jax.experimental.pallas import tpu_sc as plsc

B = 16384
F = 26
D = 16
N_FLAT = B * F
HID1, HID2 = 256, 128
IN_DIM = F * D

NC, NS = 2, 16
NW = NC * NS
PER_W = N_FLAT // NW
CHUNK = 1664
NCHUNK = PER_W // CHUNK
G = CHUNK // 128
IDX_ROWS = N_FLAT // 128

_OFF_PATTERN = (
    (np.arange(CHUNK, dtype=np.int64) % F) * 100000
).astype(np.int32)


def _gather_body(idx_hbm, offp_hbm, table_hbm, out_hbm,
                 idx_v, offp_v, rows_v, sem):
    wid = lax.axis_index("s") * NC + lax.axis_index("c")
    pltpu.sync_copy(offp_hbm, offp_v)

    def chunk_body(c, _):
        base = wid * PER_W + c * CHUNK
        pltpu.sync_copy(idx_hbm.at[pl.ds(base, CHUNK)], idx_v)
        for j in range(CHUNK // 16):
            s = pl.ds(j * 16, 16)
            idx_v[s] = idx_v[s] + offp_v[s]
        copies = []
        for g in range(G):
            copies.append(pltpu.async_copy(
                table_hbm.at[idx_v.at[pl.ds(g * 128, 128)]],
                rows_v.at[pl.ds(g * 128, 128)],
                sem))
        for cp in copies:
            cp.wait()
        pltpu.sync_copy(rows_v, out_hbm.at[pl.ds(base, CHUNK)])
        return 0

    lax.fori_loop(0, NCHUNK, chunk_body, 0)


def _sc_gather(idx2d, table):
    mesh = plsc.VectorSubcoreMesh(core_axis_name="c", subcore_axis_name="s")
    k = functools.partial(
        pl.kernel,
        mesh=mesh,
        compiler_params=pltpu.CompilerParams(use_tc_tiling_on_sc=False),
        out_type=jax.ShapeDtypeStruct((N_FLAT, D), jnp.float32),
        scratch_types=[
            pltpu.VMEM((CHUNK,), jnp.int32),
            pltpu.VMEM((CHUNK,), jnp.int32),
            pltpu.VMEM((CHUNK, D), jnp.float32),
            pltpu.SemaphoreType.DMA,
        ],
    )(_gather_body)
    return k(idx2d, jnp.asarray(_OFF_PATTERN), table)


def _mlp_body(h_ref, w1_ref, b1_ref, w2_ref, b2_ref, wo_ref, bo_ref, o_ref):
    h = h_ref[...]
    h1 = jnp.maximum(
        jnp.dot(h, w1_ref[...], preferred_element_type=jnp.float32)
        + b1_ref[...], 0.0)
    h2 = jnp.maximum(
        jnp.dot(h1, w2_ref[...], preferred_element_type=jnp.float32)
        + b2_ref[...], 0.0)
    logit = jnp.dot(h2, wo_ref[...],
                    preferred_element_type=jnp.float32)[:, 0] + bo_ref[...]
    o_ref[...] = jax.nn.sigmoid(logit)


def _tc_mlp(embed, W1, b1, W2, b2, Wo, bo):
    BLK = 2048
    grid = (B // BLK,)
    return pl.pallas_call(
        _mlp_body,
        grid=grid,
        in_specs=[
            pl.BlockSpec((BLK, IN_DIM), lambda i: (i, 0)),
            pl.BlockSpec((IN_DIM, HID1), lambda i: (0, 0)),
            pl.BlockSpec((HID1,), lambda i: (0,)),
            pl.BlockSpec((HID1, HID2), lambda i: (0, 0)),
            pl.BlockSpec((HID2,), lambda i: (0,)),
            pl.BlockSpec((HID2, 1), lambda i: (0, 0)),
            pl.BlockSpec((1,), lambda i: (0,)),
        ],
        out_specs=pl.BlockSpec((BLK,), lambda i: (i,)),
        out_shape=jax.ShapeDtypeStruct((B,), jnp.float32),
    )(embed, W1, b1, W2, b2, Wo, bo)


def kernel(x, table, W1, b1, W2, b2, Wo, bo):
    idx_flat = x.astype(jnp.int32).reshape(N_FLAT)
    rows = _sc_gather(idx_flat, table)
    embed = rows.reshape(B, IN_DIM)
    return _tc_mlp(embed, W1, b1, W2, b2, Wo, bo)
